# Initial kernel scaffold; baseline (speedup 1.0000x reference)
import jax
import jax.numpy as jnp
from jax.experimental import pallas as pl


def kernel(x, edge_index, batch, w_embed, b_embed, W1, B1, W2, B2, w_head, b_head):
    raise NotImplementedError("write your pallas kernel here")


# speedup vs baseline: 3.5245x; 3.5245x over previous
import functools

import jax
import jax.numpy as jnp
from jax import lax
from jax.experimental import pallas as pl
from jax.experimental.pallas import tpu as pltpu
from jax.experimental.pallas import tpu_sc as plsc

N = 10000
E = 640000
C = 128
L = 4
G = 128
IN_DIM = 118

NP = 10240
TILES = 32
STRIPE = NP // 16
CH = 128
CPT = 160
GSZ = 16
EP = TILES * CPT * CH

_MESH = plsc.VectorSubcoreMesh(
    core_axis_name="c", subcore_axis_name="s", num_cores=2, num_subcores=16)


@functools.partial(
    pl.kernel,
    out_type=jax.ShapeDtypeStruct((2, NP, C), jnp.float32),
    mesh=_MESH,
    scratch_types=[
        pltpu.VMEM((GSZ, CH), jnp.int32),
        pltpu.VMEM((GSZ, CH), jnp.int32),
        pltpu.VMEM((CH, C), jnp.float32),
        pltpu.VMEM_SHARED((NP, C), jnp.float32),
        pltpu.SemaphoreType.DMA,
    ],
)
def _edge_agg(m_hbm, src_hbm, dst_hbm, z_hbm, out_hbm, sidx, didx, rows, acc, sem):
    c = lax.axis_index("c")
    s = lax.axis_index("s")
    wid = c * 16 + s
    pltpu.sync_copy(z_hbm.at[pl.ds(s * STRIPE, STRIPE)],
                    acc.at[pl.ds(s * STRIPE, STRIPE)])
    plsc.subcore_barrier()

    def body(og, carry):
        pltpu.sync_copy(src_hbm.at[wid, pl.ds(og * GSZ, GSZ)], sidx)
        pltpu.sync_copy(dst_hbm.at[wid, pl.ds(og * GSZ, GSZ)], didx)
        for j in range(GSZ):
            pltpu.async_copy(m_hbm.at[sidx.at[j]], rows, sem).wait()
            pltpu.sync_copy(rows, acc.at[didx.at[j]], add=True)
        return carry

    lax.fori_loop(0, CPT // GSZ, body, 0)
    plsc.subcore_barrier()
    pltpu.sync_copy(acc.at[pl.ds(s * STRIPE, STRIPE)],
                    out_hbm.at[c, pl.ds(s * STRIPE, STRIPE)])


@functools.partial(
    pl.kernel,
    out_type=jax.ShapeDtypeStruct((TILES, NP), jnp.float32),
    mesh=_MESH,
    scratch_types=[
        pltpu.VMEM((CPT * CH,), jnp.int32),
        pltpu.VMEM((NP,), jnp.float32),
    ],
    compiler_params=pltpu.CompilerParams(needs_layout_passes=False),
)
def _deg_counts(dst_hbm, out_hbm, didx, cnt):
    c = lax.axis_index("c")
    s = lax.axis_index("s")
    wid = c * 16 + s
    pltpu.sync_copy(dst_hbm.at[wid], didx)

    def zero_body(i, carry):
        cnt[pl.ds(i * 16, 16)] = jnp.zeros((16,), jnp.float32)
        return carry

    lax.fori_loop(0, NP // 16, zero_body, 0)
    ones16 = jnp.ones((16,), jnp.float32)

    def body(t, carry):
        idx = didx[pl.ds(t * 16, 16)]
        plsc.addupdate_scatter(cnt, [idx], ones16)
        return carry

    lax.fori_loop(0, (CPT * CH) // 16, body, 0)
    pltpu.sync_copy(cnt, out_hbm.at[wid])


_BR = 1280

_INV_SQRT2 = 0.7071067811865476


def _gelu(y):
    return 0.5 * y * (1.0 + lax.erf(y * _INV_SQRT2))


def _affine_call(xp, w, b, act):
    def body(x_ref, w_ref, b_ref, o_ref):
        y = jnp.dot(x_ref[...], w_ref[...], preferred_element_type=jnp.float32)
        y = y + b_ref[...]
        o_ref[...] = act(y)

    return pl.pallas_call(
        body,
        grid=(NP // _BR,),
        in_specs=[
            pl.BlockSpec((_BR, C), lambda i: (i, 0)),
            pl.BlockSpec((C, C), lambda i: (0, 0)),
            pl.BlockSpec((1, C), lambda i: (0, 0)),
        ],
        out_specs=pl.BlockSpec((_BR, C), lambda i: (i, 0)),
        out_shape=jax.ShapeDtypeStruct((NP, C), jnp.float32),
    )(xp, w, b.reshape(1, C))


def _update_call(parts, cnt_parts, w, b):
    def body(s0_ref, s1_ref, c_ref, w_ref, b_ref, o_ref):
        cnt = lax.dot_general(
            c_ref[...], jnp.ones((TILES, 1), jnp.float32),
            (((0,), (0,)), ((), ())),
            preferred_element_type=jnp.float32)
        inv = 1.0 / jnp.maximum(cnt, 1.0)
        agg = (s0_ref[...] + s1_ref[...]) * inv
        y = jnp.dot(agg, w_ref[...], preferred_element_type=jnp.float32)
        o_ref[...] = _gelu(y + b_ref[...])

    return pl.pallas_call(
        body,
        grid=(NP // _BR,),
        in_specs=[
            pl.BlockSpec((_BR, C), lambda i: (i, 0)),
            pl.BlockSpec((_BR, C), lambda i: (i, 0)),
            pl.BlockSpec((TILES, _BR), lambda i: (0, i)),
            pl.BlockSpec((C, C), lambda i: (0, 0)),
            pl.BlockSpec((1, C), lambda i: (0, 0)),
        ],
        out_specs=pl.BlockSpec((_BR, C), lambda i: (i, 0)),
        out_shape=jax.ShapeDtypeStruct((NP, C), jnp.float32),
    )(parts[0], parts[1], cnt_parts, w, b.reshape(1, C))


def _pool_head_call(h, batch_pad, w_head, b_head):
    def body(h_ref, b_ref, wh_ref, bh_ref, o_ref):
        gids = b_ref[...]
        iota = lax.broadcasted_iota(jnp.int32, (G, NP), 0)
        onehot = (iota == gids).astype(jnp.float32)
        psum = jnp.dot(onehot, h_ref[...], preferred_element_type=jnp.float32)
        cnt = jnp.sum(onehot, axis=1, keepdims=True)
        pooled = psum / jnp.maximum(cnt, 1.0)
        o_ref[...] = jnp.dot(pooled, wh_ref[...],
                             preferred_element_type=jnp.float32) + bh_ref[...]

    return pl.pallas_call(
        body,
        out_shape=jax.ShapeDtypeStruct((G, 1), jnp.float32),
    )(h, batch_pad, w_head, b_head.reshape(1, 1))


def kernel(x, edge_index, batch, w_embed, b_embed, W1, B1, W2, B2, w_head, b_head):
    src = edge_index[0]
    dst = edge_index[1]
    pad_fill = jnp.full((EP - E,), N, jnp.int32)
    srcp = jnp.concatenate([src, pad_fill]).reshape(TILES, CPT, CH)
    dstp = jnp.concatenate([dst, pad_fill]).reshape(TILES, CPT, CH)
    dstp_flat = dstp.reshape(TILES, CPT * CH)

    x_pad = jnp.pad(x, ((0, NP - N), (0, C - IN_DIM)))
    we_pad = jnp.pad(w_embed, ((0, C - IN_DIM), (0, 0)))
    zeros_np = jnp.zeros((NP, C), jnp.float32)
    batch_pad = jnp.concatenate(
        [batch, jnp.full((NP - N,), G, jnp.int32)]).reshape(1, NP)

    cnt_parts = _deg_counts(dstp_flat)

    h = _affine_call(x_pad, we_pad, b_embed, lambda y: y)
    for l in range(L):
        m = _affine_call(h, W1[l], B1[l], _gelu)
        parts = _edge_agg(m, srcp, dstp, zeros_np)
        h = _update_call(parts, cnt_parts, W2[l], B2[l])

    return _pool_head_call(h, batch_pad, w_head, b_head)


# speedup vs baseline: 3.8143x; 1.0822x over previous
import functools

import jax
import jax.numpy as jnp
from jax import lax
from jax.experimental import pallas as pl
from jax.experimental.pallas import tpu as pltpu
from jax.experimental.pallas import tpu_sc as plsc

N = 10000
E = 640000
C = 128
L = 4
G = 128
IN_DIM = 118

NP = 10240
TILES = 32
STRIPE = NP // 16
CH = 128
CPT = 160
GSZ = 16
EP = TILES * CPT * CH

_MESH = plsc.VectorSubcoreMesh(
    core_axis_name="c", subcore_axis_name="s", num_cores=2, num_subcores=16)


@functools.partial(
    pl.kernel,
    out_type=jax.ShapeDtypeStruct((2, NP, C), jnp.float32),
    mesh=_MESH,
    scratch_types=[
        pltpu.VMEM((2, GSZ, CH), jnp.int32),
        pltpu.VMEM((2, GSZ, CH), jnp.int32),
        pltpu.VMEM((2, CH, C), jnp.float32),
        pltpu.VMEM_SHARED((NP, C), jnp.float32),
        pltpu.SemaphoreType.DMA,
    ],
)
def _edge_agg(m_hbm, src_hbm, dst_hbm, z_hbm, out_hbm, sidx, didx, rows, acc, sem):
    c = lax.axis_index("c")
    s = lax.axis_index("s")
    wid = c * 16 + s
    NG = CPT // GSZ
    pltpu.sync_copy(z_hbm.at[pl.ds(s * STRIPE, STRIPE)],
                    acc.at[pl.ds(s * STRIPE, STRIPE)])
    plsc.subcore_barrier()

    def stage(buf, grp):
        pltpu.sync_copy(src_hbm.at[wid, pl.ds(grp * GSZ, GSZ)], sidx.at[buf])
        pltpu.sync_copy(dst_hbm.at[wid, pl.ds(grp * GSZ, GSZ)], didx.at[buf])

    stage(0, 0)
    pltpu.async_copy(m_hbm.at[sidx.at[0].at[0]], rows.at[0], sem)

    def body(og, carry):
        q = og & 1
        stage(1 - q, jnp.minimum(og + 1, NG - 1))
        for j in range(GSZ):
            p = j & 1
            pltpu.make_async_copy(m_hbm.at[sidx.at[q].at[j]],
                                  rows.at[p], sem).wait()
            if j + 1 < GSZ:
                pltpu.async_copy(m_hbm.at[sidx.at[q].at[j + 1]],
                                 rows.at[1 - p], sem)
            else:
                pltpu.async_copy(m_hbm.at[sidx.at[1 - q].at[0]],
                                 rows.at[1 - p], sem)
            pltpu.sync_copy(rows.at[p], acc.at[didx.at[q].at[j]], add=True)
        return carry

    lax.fori_loop(0, NG, body, 0)
    pltpu.make_async_copy(m_hbm.at[sidx.at[0].at[0]], rows.at[0], sem).wait()
    plsc.subcore_barrier()
    pltpu.sync_copy(acc.at[pl.ds(s * STRIPE, STRIPE)],
                    out_hbm.at[c, pl.ds(s * STRIPE, STRIPE)])


@functools.partial(
    pl.kernel,
    out_type=jax.ShapeDtypeStruct((TILES, NP), jnp.float32),
    mesh=_MESH,
    scratch_types=[
        pltpu.VMEM((CPT * CH,), jnp.int32),
        pltpu.VMEM((NP,), jnp.float32),
    ],
    compiler_params=pltpu.CompilerParams(needs_layout_passes=False),
)
def _deg_counts(dst_hbm, out_hbm, didx, cnt):
    c = lax.axis_index("c")
    s = lax.axis_index("s")
    wid = c * 16 + s
    pltpu.sync_copy(dst_hbm.at[wid], didx)

    def zero_body(i, carry):
        cnt[pl.ds(i * 16, 16)] = jnp.zeros((16,), jnp.float32)
        return carry

    lax.fori_loop(0, NP // 16, zero_body, 0)
    ones16 = jnp.ones((16,), jnp.float32)

    def body(t, carry):
        idx = didx[pl.ds(t * 16, 16)]
        plsc.addupdate_scatter(cnt, [idx], ones16)
        return carry

    lax.fori_loop(0, (CPT * CH) // 16, body, 0)
    pltpu.sync_copy(cnt, out_hbm.at[wid])


_BR = 1280

_INV_SQRT2 = 0.7071067811865476


def _gelu(y):
    return 0.5 * y * (1.0 + lax.erf(y * _INV_SQRT2))


def _affine_call(xp, w, b, act):
    def body(x_ref, w_ref, b_ref, o_ref):
        y = jnp.dot(x_ref[...], w_ref[...], preferred_element_type=jnp.float32)
        y = y + b_ref[...]
        o_ref[...] = act(y)

    return pl.pallas_call(
        body,
        grid=(NP // _BR,),
        in_specs=[
            pl.BlockSpec((_BR, C), lambda i: (i, 0)),
            pl.BlockSpec((C, C), lambda i: (0, 0)),
            pl.BlockSpec((1, C), lambda i: (0, 0)),
        ],
        out_specs=pl.BlockSpec((_BR, C), lambda i: (i, 0)),
        out_shape=jax.ShapeDtypeStruct((NP, C), jnp.float32),
    )(xp, w, b.reshape(1, C))


def _update_call(parts, cnt_parts, w, b):
    def body(s0_ref, s1_ref, c_ref, w_ref, b_ref, o_ref):
        cnt = lax.dot_general(
            c_ref[...], jnp.ones((TILES, 1), jnp.float32),
            (((0,), (0,)), ((), ())),
            preferred_element_type=jnp.float32)
        inv = 1.0 / jnp.maximum(cnt, 1.0)
        agg = (s0_ref[...] + s1_ref[...]) * inv
        y = jnp.dot(agg, w_ref[...], preferred_element_type=jnp.float32)
        o_ref[...] = _gelu(y + b_ref[...])

    return pl.pallas_call(
        body,
        grid=(NP // _BR,),
        in_specs=[
            pl.BlockSpec((_BR, C), lambda i: (i, 0)),
            pl.BlockSpec((_BR, C), lambda i: (i, 0)),
            pl.BlockSpec((TILES, _BR), lambda i: (0, i)),
            pl.BlockSpec((C, C), lambda i: (0, 0)),
            pl.BlockSpec((1, C), lambda i: (0, 0)),
        ],
        out_specs=pl.BlockSpec((_BR, C), lambda i: (i, 0)),
        out_shape=jax.ShapeDtypeStruct((NP, C), jnp.float32),
    )(parts[0], parts[1], cnt_parts, w, b.reshape(1, C))


def _pool_head_call(h, batch_pad, w_head, b_head):
    def body(h_ref, b_ref, wh_ref, bh_ref, o_ref):
        gids = b_ref[...]
        iota = lax.broadcasted_iota(jnp.int32, (G, NP), 0)
        onehot = (iota == gids).astype(jnp.float32)
        psum = jnp.dot(onehot, h_ref[...], preferred_element_type=jnp.float32)
        cnt = jnp.sum(onehot, axis=1, keepdims=True)
        pooled = psum / jnp.maximum(cnt, 1.0)
        o_ref[...] = jnp.dot(pooled, wh_ref[...],
                             preferred_element_type=jnp.float32) + bh_ref[...]

    return pl.pallas_call(
        body,
        out_shape=jax.ShapeDtypeStruct((G, 1), jnp.float32),
    )(h, batch_pad, w_head, b_head.reshape(1, 1))


def kernel(x, edge_index, batch, w_embed, b_embed, W1, B1, W2, B2, w_head, b_head):
    src = edge_index[0]
    dst = edge_index[1]
    pad_fill = jnp.full((EP - E,), N, jnp.int32)
    srcp = jnp.concatenate([src, pad_fill]).reshape(TILES, CPT, CH)
    dstp = jnp.concatenate([dst, pad_fill]).reshape(TILES, CPT, CH)
    dstp_flat = dstp.reshape(TILES, CPT * CH)

    x_pad = jnp.pad(x, ((0, NP - N), (0, C - IN_DIM)))
    we_pad = jnp.pad(w_embed, ((0, C - IN_DIM), (0, 0)))
    zeros_np = jnp.zeros((NP, C), jnp.float32)
    batch_pad = jnp.concatenate(
        [batch, jnp.full((NP - N,), G, jnp.int32)]).reshape(1, NP)

    cnt_parts = _deg_counts(dstp_flat)

    h = _affine_call(x_pad, we_pad, b_embed, lambda y: y)
    for l in range(L):
        m = _affine_call(h, W1[l], B1[l], _gelu)
        parts = _edge_agg(m, srcp, dstp, zeros_np)
        h = _update_call(parts, cnt_parts, W2[l], B2[l])

    return _pool_head_call(h, batch_pad, w_head, b_head)
